# baseline (speedup 1.0000x reference)
import functools

import jax
import jax.numpy as jnp
from jax.experimental import pallas as pl
from jax.experimental.pallas import tpu as pltpu


@functools.lru_cache(maxsize=None)
def _build_tc_kernel(B, H, W, C, CB):
    NBLK = 2 * C // CB
    NCB = C // CB

    def body(colT_ref, rowT_ref, out_hbm, scratch, sem):
        b = pl.program_id(0)
        i = pl.program_id(1)

        @pl.when(b == 0)
        def _build():
            @pl.when(i < NCB)
            def _():
                scratch[pl.ds(i * CB, CB)] = jnp.broadcast_to(
                    colT_ref[...][:, None, :], (CB, H, W)
                )

            @pl.when(i >= NCB)
            def _():
                scratch[pl.ds(i * CB, CB)] = jnp.broadcast_to(
                    rowT_ref[...][:, :, None], (CB, H, W)
                )

        cp = pltpu.make_async_copy(
            scratch.at[pl.ds(i * CB, CB)],
            out_hbm.at[b, pl.ds(i * CB, CB)],
            sem,
        )
        cp.start()
        cp.wait()

    grid = (B, NBLK)
    return pl.pallas_call(
        body,
        grid=grid,
        in_specs=[
            pl.BlockSpec((CB, W), lambda b, i: (i % NCB, 0)),
            pl.BlockSpec((CB, H), lambda b, i: (i % NCB, 0)),
        ],
        out_specs=pl.BlockSpec(memory_space=pl.ANY),
        out_shape=jax.ShapeDtypeStruct((B, 2 * C, H, W), jnp.float32),
        scratch_shapes=[
            pltpu.VMEM((2 * C, H, W), jnp.float32),
            pltpu.SemaphoreType.DMA,
        ],
        compiler_params=pltpu.CompilerParams(
            dimension_semantics=("arbitrary", "arbitrary"),
        ),
    )


def kernel(mask, row_embed, col_embed):
    B, H, W = mask.shape
    C = col_embed.shape[1]
    colT = col_embed.T
    rowT = row_embed.T
    return _build_tc_kernel(B, H, W, C, 64)(colT, rowT)


# speedup vs baseline: 6.7664x; 6.7664x over previous
import functools

import jax
import jax.numpy as jnp
from jax.experimental import pallas as pl
from jax.experimental.pallas import tpu as pltpu


@functools.lru_cache(maxsize=None)
def _build_tc_kernel(B, H, W, C):
    def body(colB_ref, rowB_ref, out_ref):
        out_ref[0, :, :, 0:C] = jnp.broadcast_to(colB_ref[...], (W, B, C))
        out_ref[0, :, :, C : 2 * C] = jnp.broadcast_to(rowB_ref[...], (W, B, C))

    return pl.pallas_call(
        body,
        grid=(H,),
        in_specs=[
            pl.BlockSpec((W, 1, C), lambda h: (0, 0, 0)),
            pl.BlockSpec((1, 1, C), lambda h: (h, 0, 0)),
        ],
        out_specs=pl.BlockSpec((1, W, B, 2 * C), lambda h: (h, 0, 0, 0)),
        out_shape=jax.ShapeDtypeStruct((H, W, B, 2 * C), jnp.float32),
        compiler_params=pltpu.CompilerParams(
            dimension_semantics=("arbitrary",),
        ),
    )


def kernel(mask, row_embed, col_embed):
    B, H, W = mask.shape
    C = col_embed.shape[1]
    colB = col_embed.reshape(W, 1, C)
    rowB = row_embed.reshape(H, 1, C)
    out_hwbc = _build_tc_kernel(B, H, W, C)(colB, rowB)
    return jnp.transpose(out_hwbc, (2, 3, 0, 1))


# speedup vs baseline: 9.1579x; 1.3534x over previous
import functools

import jax
import jax.numpy as jnp
from jax.experimental import pallas as pl
from jax.experimental.pallas import tpu as pltpu


@functools.lru_cache(maxsize=None)
def _build_tc_kernel(B, H, W, C, HB):
    def body(colB_ref, rowB_ref, out_ref):
        out_ref[:, :, :, 0:C] = jnp.broadcast_to(colB_ref[...], (HB, W, B, C))
        out_ref[:, :, :, C : 2 * C] = jnp.broadcast_to(
            rowB_ref[...], (HB, W, B, C)
        )

    return pl.pallas_call(
        body,
        grid=(H // HB,),
        in_specs=[
            pl.BlockSpec((1, W, 1, C), lambda h: (0, 0, 0, 0)),
            pl.BlockSpec((HB, 1, 1, C), lambda h: (h, 0, 0, 0)),
        ],
        out_specs=pl.BlockSpec((HB, W, B, 2 * C), lambda h: (h, 0, 0, 0)),
        out_shape=jax.ShapeDtypeStruct((H, W, B, 2 * C), jnp.float32),
        compiler_params=pltpu.CompilerParams(
            dimension_semantics=("arbitrary",),
        ),
    )


def kernel(mask, row_embed, col_embed):
    B, H, W = mask.shape
    C = col_embed.shape[1]
    colB = col_embed.reshape(1, W, 1, C)
    rowB = row_embed.reshape(H, 1, 1, C)
    out_hwbc = _build_tc_kernel(B, H, W, C, 5)(colB, rowB)
    return jnp.transpose(out_hwbc, (2, 3, 0, 1))
